# Initial kernel scaffold; baseline (speedup 1.0000x reference)
import jax
import jax.numpy as jnp
from jax.experimental import pallas as pl


def kernel(x, table):
    raise NotImplementedError("write your pallas kernel here")


# speedup vs baseline: 1.1106x; 1.1106x over previous
import functools

import jax
import jax.numpy as jnp
from jax import lax
from jax.experimental import pallas as pl
from jax.experimental.pallas import tpu as pltpu
from jax.experimental.pallas import tpu_sc as plsc

_NUM_CORES = 2
_NUM_SUBCORES = 16
_NW = _NUM_CORES * _NUM_SUBCORES
_CHUNK = 3200


@functools.lru_cache(maxsize=None)
def _make_gather(total_rows: int, vocab: int, dim: int):
    b_per_w = total_rows // _NW
    n_chunks = b_per_w // _CHUNK
    assert n_chunks * _CHUNK == b_per_w
    mesh = plsc.VectorSubcoreMesh(core_axis_name="c", subcore_axis_name="s")

    @functools.partial(
        pl.kernel,
        mesh=mesh,
        out_type=jax.ShapeDtypeStruct((total_rows, dim), jnp.float32),
        compiler_params=pltpu.CompilerParams(use_tc_tiling_on_sc=False),
        scratch_types=[
            pltpu.VMEM((_CHUNK,), jnp.int32),
            pltpu.VMEM((_CHUNK, dim), jnp.float32),
            pltpu.SemaphoreType.DMA,
        ],
    )
    def gather_kernel(idx_hbm, table_hbm, out_hbm, idx_v, rows_v, sem):
        wid = lax.axis_index("s") * _NUM_CORES + lax.axis_index("c")
        base = wid * b_per_w
        for g in range(n_chunks):
            off = base + g * _CHUNK
            pltpu.sync_copy(idx_hbm.at[pl.ds(off, _CHUNK)], idx_v)
            pltpu.async_copy(table_hbm.at[idx_v], rows_v, sem).wait()
            pltpu.sync_copy(rows_v, out_hbm.at[pl.ds(off, _CHUNK)])

    return gather_kernel


def kernel(x, table):
    batch, hist = x.shape
    vocab, dim = table.shape
    idx = x.reshape(-1).astype(jnp.int32)
    out = _make_gather(batch * hist, vocab, dim)(idx, table)
    return out.reshape(batch, hist, dim)


# speedup vs baseline: 1.1140x; 1.0030x over previous
import functools

import jax
import jax.numpy as jnp
from jax import lax
from jax.experimental import pallas as pl
from jax.experimental.pallas import tpu as pltpu
from jax.experimental.pallas import tpu_sc as plsc

_NUM_CORES = 2
_NUM_SUBCORES = 16
_NW = _NUM_CORES * _NUM_SUBCORES
_CHUNK = 1600


@functools.lru_cache(maxsize=None)
def _make_gather(total_rows: int, vocab: int, dim: int):
    b_per_w = total_rows // _NW
    n_chunks = b_per_w // _CHUNK
    assert n_chunks * _CHUNK == b_per_w
    mesh = plsc.VectorSubcoreMesh(core_axis_name="c", subcore_axis_name="s")

    @functools.partial(
        pl.kernel,
        mesh=mesh,
        out_type=jax.ShapeDtypeStruct((total_rows, dim), jnp.float32),
        compiler_params=pltpu.CompilerParams(use_tc_tiling_on_sc=False),
        scratch_types=[
            pltpu.VMEM((b_per_w,), jnp.int32),
            pltpu.VMEM((_CHUNK, dim), jnp.float32),
            pltpu.VMEM((_CHUNK, dim), jnp.float32),
            pltpu.SemaphoreType.DMA,
            pltpu.SemaphoreType.DMA,
            pltpu.SemaphoreType.DMA,
            pltpu.SemaphoreType.DMA,
        ],
    )
    def gather_kernel(idx_hbm, table_hbm, out_hbm, idx_v, r0, r1, gs0, gs1, ss0, ss1):
        rows = (r0, r1)
        gsem = (gs0, gs1)
        ssem = (ss0, ss1)
        wid = lax.axis_index("s") * _NUM_CORES + lax.axis_index("c")
        base = wid * b_per_w
        pltpu.sync_copy(idx_hbm.at[pl.ds(base, b_per_w)], idx_v)

        def start_gather(g):
            b = g % 2
            return pltpu.async_copy(
                table_hbm.at[idx_v.at[pl.ds(g * _CHUNK, _CHUNK)]], rows[b], gsem[b])

        def start_store(g):
            b = g % 2
            return pltpu.async_copy(
                rows[b], out_hbm.at[pl.ds(base + g * _CHUNK, _CHUNK)], ssem[b])

        gh = [None] * n_chunks
        sh = [None] * n_chunks
        gh[0] = start_gather(0)
        for g in range(n_chunks):
            if g + 1 < n_chunks:
                if g >= 1:
                    sh[g - 1].wait()
                gh[g + 1] = start_gather(g + 1)
            gh[g].wait()
            sh[g] = start_store(g)
        sh[n_chunks - 1].wait()
        if n_chunks >= 2:
            sh[n_chunks - 2].wait()

    return gather_kernel


def kernel(x, table):
    batch, hist = x.shape
    vocab, dim = table.shape
    idx = x.reshape(-1).astype(jnp.int32)
    out = _make_gather(batch * hist, vocab, dim)(idx, table)
    return out.reshape(batch, hist, dim)


# speedup vs baseline: 1.8058x; 1.6210x over previous
import functools

import jax
import jax.numpy as jnp
from jax import lax
from jax.experimental import pallas as pl
from jax.experimental.pallas import tpu as pltpu
from jax.experimental.pallas import tpu_sc as plsc

_NUM_CORES = 2
_NUM_SUBCORES = 16
_NW = _NUM_CORES * _NUM_SUBCORES
_CB = 32


@functools.lru_cache(maxsize=None)
def _make_gather(batch: int, hist: int, vocab: int, dim: int):
    b_per_w = batch // _NW
    chunk_rows = _CB * hist
    n_chunks = b_per_w // _CB
    n_total_chunks = batch // _CB
    assert n_chunks * _CB == b_per_w
    mesh = plsc.VectorSubcoreMesh(core_axis_name="c", subcore_axis_name="s")

    @functools.partial(
        pl.kernel,
        mesh=mesh,
        out_type=jax.ShapeDtypeStruct((batch, hist, dim), jnp.float32),
        compiler_params=pltpu.CompilerParams(use_tc_tiling_on_sc=False),
        scratch_types=[
            pltpu.VMEM((b_per_w, hist), jnp.int32),
            pltpu.VMEM((_CB, hist, dim), jnp.float32),
            pltpu.VMEM((_CB, hist, dim), jnp.float32),
            pltpu.SemaphoreType.DMA,
            pltpu.SemaphoreType.DMA,
            pltpu.SemaphoreType.DMA,
            pltpu.SemaphoreType.DMA,
        ],
    )
    def gather_kernel(x_hbm, table_hbm, out_hbm, idx_v, r0, r1, gs0, gs1, ss0, ss1):
        rows = (r0, r1)
        gsem = (gs0, gs1)
        ssem = (ss0, ss1)
        wid = lax.axis_index("s") * _NUM_CORES + lax.axis_index("c")
        base_b = wid * b_per_w
        pltpu.sync_copy(x_hbm.at[pl.ds(base_b, b_per_w)], idx_v)

        def start_gather(g):
            b = g % 2
            return [
                pltpu.async_copy(
                    table_hbm.at[idx_v.at[g * _CB + j]], rows[b].at[j], gsem[b])
                for j in range(_CB)
            ]

        def start_store(g):
            b = g % 2
            return pltpu.async_copy(
                rows[b], out_hbm.at[pl.ds(base_b + g * _CB, _CB)], ssem[b])

        gh = [None] * n_chunks
        sh = [None] * n_chunks
        gh[0] = start_gather(0)
        for g in range(n_chunks):
            if g + 1 < n_chunks:
                if g >= 1:
                    sh[g - 1].wait()
                gh[g + 1] = start_gather(g + 1)
            for h in gh[g]:
                h.wait()
            sh[g] = start_store(g)
        sh[n_chunks - 1].wait()
        if n_chunks >= 2:
            sh[n_chunks - 2].wait()

    return gather_kernel


def kernel(x, table):
    batch, hist = x.shape
    vocab, dim = table.shape
    return _make_gather(batch, hist, vocab, dim)(x.astype(jnp.int32), table)
